# initial kernel scaffold (unmeasured)
import jax
import jax.numpy as jnp
from jax import lax
from jax.experimental import pallas as pl
from jax.experimental.pallas import tpu as pltpu


def kernel(
    x,
):
    def body(*refs):
        pass

    out_shape = jax.ShapeDtypeStruct(..., jnp.float32)
    return pl.pallas_call(body, out_shape=out_shape)(...)


# baseline (device time: 52241 ns/iter reference)
import jax
import jax.numpy as jnp
from jax import lax
from jax.experimental import pallas as pl
from jax.experimental.pallas import tpu as pltpu

N_DEV = 4


def kernel(x):
    m, n = x.shape
    m_chunk = m // N_DEV

    def body(x_ref, out_ref, comm_ref, send_sems, recv_sems):
        my_x = lax.axis_index("x")
        my_y = lax.axis_index("y")
        my_z = lax.axis_index("z")
        left = (my_y - 1) % N_DEV
        right = (my_y + 1) % N_DEV

        barrier_sem = pltpu.get_barrier_semaphore()
        for nbr in (left, right):
            pl.semaphore_signal(
                barrier_sem,
                inc=1,
                device_id=(my_x, nbr, my_z),
                device_id_type=pl.DeviceIdType.MESH,
            )
        pl.semaphore_wait(barrier_sem, 2)

        out_ref[:, :] = x_ref[:, :]

        for s in range(N_DEV - 1):
            send_c = (my_y - s) % N_DEV
            recv_c = (my_y - s - 1) % N_DEV
            rdma = pltpu.make_async_remote_copy(
                src_ref=out_ref.at[pl.ds(send_c * m_chunk, m_chunk), :],
                dst_ref=comm_ref.at[s],
                send_sem=send_sems.at[s],
                recv_sem=recv_sems.at[s],
                device_id=(my_x, right, my_z),
                device_id_type=pl.DeviceIdType.MESH,
            )
            rdma.start()
            rdma.wait()
            out_ref[pl.ds(recv_c * m_chunk, m_chunk), :] = (
                out_ref[pl.ds(recv_c * m_chunk, m_chunk), :] + comm_ref[s]
            )

        for s in range(N_DEV - 1):
            c = (my_y + 1 - s) % N_DEV
            rdma = pltpu.make_async_remote_copy(
                src_ref=out_ref.at[pl.ds(c * m_chunk, m_chunk), :],
                dst_ref=out_ref.at[pl.ds(c * m_chunk, m_chunk), :],
                send_sem=send_sems.at[N_DEV - 1 + s],
                recv_sem=recv_sems.at[N_DEV - 1 + s],
                device_id=(my_x, right, my_z),
                device_id_type=pl.DeviceIdType.MESH,
            )
            rdma.start()
            rdma.wait()

    n_steps = 2 * (N_DEV - 1)
    return pl.pallas_call(
        body,
        out_shape=jax.ShapeDtypeStruct((m, n), x.dtype),
        in_specs=[pl.BlockSpec(memory_space=pltpu.VMEM)],
        out_specs=pl.BlockSpec(memory_space=pltpu.VMEM),
        scratch_shapes=[
            pltpu.VMEM((N_DEV - 1, m_chunk, n), x.dtype),
            pltpu.SemaphoreType.DMA((n_steps,)),
            pltpu.SemaphoreType.DMA((n_steps,)),
        ],
        compiler_params=pltpu.CompilerParams(collective_id=0),
    )(x)


# device time: 42082 ns/iter; 1.2414x vs baseline; 1.2414x over previous
import jax
import jax.numpy as jnp
from jax import lax
from jax.experimental import pallas as pl
from jax.experimental.pallas import tpu as pltpu

N_DEV = 4
K = 2


def kernel(x):
    m, n = x.shape
    m_chunk = m // N_DEV
    half = m_chunk // 2
    m_sub = half // K

    def body(x_ref, out_ref, comm_ref, send_sems, recv_sems):
        my_x = lax.axis_index("x")
        my_y = lax.axis_index("y")
        my_z = lax.axis_index("z")
        left = (my_y - 1) % N_DEV
        right = (my_y + 1) % N_DEV

        def row(c, d, j):
            return c * m_chunk + d * half + j * m_sub

        def rs_rdma(d, s, j):
            if d == 0:
                sc = (my_y - s) % N_DEV
                tgt = right
            else:
                sc = (my_y + s) % N_DEV
                tgt = left
            return pltpu.make_async_remote_copy(
                src_ref=out_ref.at[pl.ds(row(sc, d, j), m_sub), :],
                dst_ref=comm_ref.at[d, s, j],
                send_sem=send_sems.at[d, s, j],
                recv_sem=recv_sems.at[d, s, j],
                device_id=(my_x, tgt, my_z),
                device_id_type=pl.DeviceIdType.MESH,
            )

        def ag_rdma(d, s, j):
            if d == 0:
                c = (my_y + 1 - s) % N_DEV
                tgt = right
            else:
                c = (my_y - 1 + s) % N_DEV
                tgt = left
            sl = pl.ds(row(c, d, j), m_sub)
            return pltpu.make_async_remote_copy(
                src_ref=out_ref.at[sl, :],
                dst_ref=out_ref.at[sl, :],
                send_sem=send_sems.at[d, N_DEV - 1 + s, j],
                recv_sem=recv_sems.at[d, N_DEV - 1 + s, j],
                device_id=(my_x, tgt, my_z),
                device_id_type=pl.DeviceIdType.MESH,
            )

        def rs_accum(d, s, j):
            if d == 0:
                rc = (my_y - s - 1) % N_DEV
            else:
                rc = (my_y + s + 1) % N_DEV
            sl = pl.ds(row(rc, d, j), m_sub)
            out_ref[sl, :] = out_ref[sl, :] + comm_ref[d, s, j]

        barrier_sem = pltpu.get_barrier_semaphore()
        for nbr in (left, right):
            pl.semaphore_signal(
                barrier_sem,
                inc=1,
                device_id=(my_x, nbr, my_z),
                device_id_type=pl.DeviceIdType.MESH,
            )
        pl.semaphore_wait(barrier_sem, 2)

        out_ref[:, :] = x_ref[:, :]

        for j in range(K):
            for d in (0, 1):
                rs_rdma(d, 0, j).start()
        for s in (1, 2):
            for j in range(K):
                for d in (0, 1):
                    rs_rdma(d, s - 1, j).wait_recv()
                    rs_accum(d, s - 1, j)
                    rs_rdma(d, s, j).start()
        for j in range(K):
            for d in (0, 1):
                rs_rdma(d, N_DEV - 2, j).wait_recv()
                rs_accum(d, N_DEV - 2, j)
                ag_rdma(d, 0, j).start()
        for s in (1, 2):
            for j in range(K):
                for d in (0, 1):
                    ag_rdma(d, s - 1, j).wait_recv()
                    ag_rdma(d, s, j).start()
        for j in range(K):
            for d in (0, 1):
                ag_rdma(d, N_DEV - 2, j).wait_recv()
        for s in range(N_DEV - 1):
            for j in range(K):
                for d in (0, 1):
                    rs_rdma(d, s, j).wait_send()
                    ag_rdma(d, s, j).wait_send()

    n_steps = 2 * (N_DEV - 1)
    return pl.pallas_call(
        body,
        out_shape=jax.ShapeDtypeStruct((m, n), x.dtype),
        in_specs=[pl.BlockSpec(memory_space=pltpu.VMEM)],
        out_specs=pl.BlockSpec(memory_space=pltpu.VMEM),
        scratch_shapes=[
            pltpu.VMEM((2, N_DEV - 1, K, m_sub, n), x.dtype),
            pltpu.SemaphoreType.DMA((2, n_steps, K)),
            pltpu.SemaphoreType.DMA((2, n_steps, K)),
        ],
        compiler_params=pltpu.CompilerParams(collective_id=0),
    )(x)
